# Initial kernel scaffold; baseline (speedup 1.0000x reference)
import jax
import jax.numpy as jnp
from jax.experimental import pallas as pl


def kernel(x, edges, W, att_src, att_dst, bias):
    raise NotImplementedError("write your pallas kernel here")


# speedup vs baseline: 25.9345x; 25.9345x over previous
import functools

import jax
import jax.numpy as jnp
from jax import lax
from jax.experimental import pallas as pl
from jax.experimental.pallas import tpu as pltpu
from jax.experimental.pallas import tpu_sc as plsc

N = 10000
E = 320000
C = 128

NC = 2
NS = 16
NW = NC * NS
EPW = E // NW
B = 80
NB = EPW // B
ZR = 1000
LG = C // 16

TC_BLK = 1000
TC_GRID = N // TC_BLK


def _proj_body(x_ref, w_ref, att2_ref, h_ref, asd_ref):
    h = jnp.dot(x_ref[...], w_ref[...], preferred_element_type=jnp.float32)
    h_ref[...] = h
    asd_ref[...] = jnp.dot(h, att2_ref[...], preferred_element_type=jnp.float32)


def _proj(x, w, att2):
    return pl.pallas_call(
        _proj_body,
        grid=(TC_GRID,),
        in_specs=[
            pl.BlockSpec((TC_BLK, C), lambda i: (i, 0)),
            pl.BlockSpec((C, C), lambda i: (0, 0)),
            pl.BlockSpec((C, 2), lambda i: (0, 0)),
        ],
        out_specs=[
            pl.BlockSpec((TC_BLK, C), lambda i: (i, 0)),
            pl.BlockSpec((TC_BLK, 2), lambda i: (i, 0)),
        ],
        out_shape=[
            jax.ShapeDtypeStruct((N, C), jnp.float32),
            jax.ShapeDtypeStruct((N, 2), jnp.float32),
        ],
    )(x, w, att2)


def _combine_temp(acc_ref, dent_ref, asd_ref, h_ref, bias_ref):
    a_s = asd_ref[:, 0:1]
    a_d = asd_ref[:, 1:2]
    es = a_s + a_d
    es = jnp.where(es >= 0, es, 0.2 * es)
    exs = jnp.exp(es)
    den = jnp.sum(dent_ref[...], axis=1, keepdims=True) + exs + 1e-16
    num = acc_ref[0] + acc_ref[1] + exs * h_ref[...]
    return num / den + bias_ref[...]


def _combine_mid_body(acc_ref, dent_ref, asd_ref, h_ref, bias_ref, w_ref,
                      att2_ref, h2_ref, asd2_ref):
    temp = _combine_temp(acc_ref, dent_ref, asd_ref, h_ref, bias_ref)
    y = jnp.where(temp >= 0, temp, 0.01 * temp) + temp
    h2 = jnp.dot(y, w_ref[...], preferred_element_type=jnp.float32)
    h2_ref[...] = h2
    asd2_ref[...] = jnp.dot(h2, att2_ref[...], preferred_element_type=jnp.float32)


def _combine_mid(acc, dent, asd, h, bias2, w, att2):
    return pl.pallas_call(
        _combine_mid_body,
        grid=(TC_GRID,),
        in_specs=[
            pl.BlockSpec((2, TC_BLK, C), lambda i: (0, i, 0)),
            pl.BlockSpec((TC_BLK, NW), lambda i: (i, 0)),
            pl.BlockSpec((TC_BLK, 2), lambda i: (i, 0)),
            pl.BlockSpec((TC_BLK, C), lambda i: (i, 0)),
            pl.BlockSpec((1, C), lambda i: (0, 0)),
            pl.BlockSpec((C, C), lambda i: (0, 0)),
            pl.BlockSpec((C, 2), lambda i: (0, 0)),
        ],
        out_specs=[
            pl.BlockSpec((TC_BLK, C), lambda i: (i, 0)),
            pl.BlockSpec((TC_BLK, 2), lambda i: (i, 0)),
        ],
        out_shape=[
            jax.ShapeDtypeStruct((N, C), jnp.float32),
            jax.ShapeDtypeStruct((N, 2), jnp.float32),
        ],
    )(acc, dent, asd, h, bias2, w, att2)


def _combine_final_body(acc_ref, dent_ref, asd_ref, h_ref, bias_ref, out_ref):
    temp = _combine_temp(acc_ref, dent_ref, asd_ref, h_ref, bias_ref)
    out_ref[...] = jnp.where(temp >= 0, temp, 0.01 * temp)


def _combine_final(acc, dent, asd, h, bias2):
    return pl.pallas_call(
        _combine_final_body,
        grid=(TC_GRID,),
        in_specs=[
            pl.BlockSpec((2, TC_BLK, C), lambda i: (0, i, 0)),
            pl.BlockSpec((TC_BLK, NW), lambda i: (i, 0)),
            pl.BlockSpec((TC_BLK, 2), lambda i: (i, 0)),
            pl.BlockSpec((TC_BLK, C), lambda i: (i, 0)),
            pl.BlockSpec((1, C), lambda i: (0, 0)),
        ],
        out_specs=pl.BlockSpec((TC_BLK, C), lambda i: (i, 0)),
        out_shape=jax.ShapeDtypeStruct((N, C), jnp.float32),
    )(acc, dent, asd, h, bias2)


_SC_MESH = plsc.VectorSubcoreMesh(core_axis_name="c", subcore_axis_name="s")


@functools.partial(
    pl.kernel,
    out_type=[
        jax.ShapeDtypeStruct((NC, N, C), jnp.float32),
        jax.ShapeDtypeStruct((NW * N,), jnp.float32),
    ],
    mesh=_SC_MESH,
    compiler_params=pltpu.CompilerParams(needs_layout_passes=False),
    scratch_types=[
        pltpu.VMEM((B,), jnp.int32),
        pltpu.VMEM((B,), jnp.int32),
        pltpu.VMEM((N,), jnp.float32),
        pltpu.VMEM((N,), jnp.float32),
        pltpu.VMEM((N,), jnp.float32),
        pltpu.VMEM((B, C), jnp.float32),
        pltpu.VMEM_SHARED((N, C), jnp.float32),
        pltpu.SemaphoreType.DMA,
        pltpu.SemaphoreType.DMA,
        pltpu.SemaphoreType.DMA,
    ],
)
def _sc_edges(h_hbm, a_s_hbm, a_d_hbm, src_hbm, dst_hbm,
              zacc_hbm, acc_out, den_out,
              src_b, dst_b, as_v, ad_v, den_v, rows_v,
              acc_sh, isem, gsem, ssem):
    c = lax.axis_index("c")
    s = lax.axis_index("s")
    wid = c * NS + s

    pltpu.sync_copy(a_s_hbm, as_v)
    pltpu.sync_copy(a_d_hbm, ad_v)

    @pl.when(s < N // ZR)
    def _():
        pltpu.sync_copy(zacc_hbm.at[pl.ds(s * ZR, ZR)],
                        acc_sh.at[pl.ds(s * ZR, ZR)])

    zero16 = jnp.zeros((16,), jnp.float32)

    @pl.loop(0, N // 16)
    def _(i):
        den_v[pl.ds(i * 16, 16)] = zero16

    plsc.subcore_barrier()

    @pl.loop(0, NB)
    def _(b):
        base = wid * EPW + b * B
        pltpu.async_copy(src_hbm.at[pl.ds(base, B)], src_b, isem).wait()
        pltpu.async_copy(dst_hbm.at[pl.ds(base, B)], dst_b, isem).wait()
        pltpu.async_copy(h_hbm.at[src_b], rows_v, gsem).wait()
        for g in range(B // 16):
            off = g * 16
            s16 = src_b[pl.ds(off, 16)]
            d16 = dst_b[pl.ds(off, 16)]
            e = plsc.load_gather(as_v, [s16]) + plsc.load_gather(ad_v, [d16])
            e = jnp.where(e >= 0, e, 0.2 * e)
            ex = jnp.exp(e)
            plsc.addupdate_scatter(den_v, [d16], ex)
            for j in range(16):
                r = g * 16 + j
                w = ex[j]
                for lg in range(LG):
                    rows_v[r, pl.ds(lg * 16, 16)] = (
                        rows_v[r, pl.ds(lg * 16, 16)] * w)
        pltpu.async_copy(rows_v, acc_sh.at[dst_b], ssem, add=True).wait()

    pltpu.sync_copy(den_v, den_out.at[pl.ds(wid * N, N)])

    plsc.subcore_barrier()

    @pl.when(s < N // ZR)
    def _():
        pltpu.sync_copy(acc_sh.at[pl.ds(s * ZR, ZR)],
                        acc_out.at[c, pl.ds(s * ZR, ZR)])


def kernel(x, edges, W, att_src, att_dst, bias):
    att2 = jnp.stack([att_src, att_dst], axis=1)
    bias2 = bias.reshape(1, C)
    src_f = edges[0]
    dst_f = edges[1]
    zacc = jnp.zeros((N, C), jnp.float32)

    h1, asd1 = _proj(x, W, att2)
    acc1, den1 = _sc_edges(h1, asd1[:, 0], asd1[:, 1], src_f, dst_f, zacc)
    h2, asd2 = _combine_mid(acc1, den1.reshape(NW, N).T, asd1, h1, bias2, W,
                            att2)
    acc2, den2 = _sc_edges(h2, asd2[:, 0], asd2[:, 1], src_f, dst_f, zacc)
    return _combine_final(acc2, den2.reshape(NW, N).T, asd2, h2, bias2)


# speedup vs baseline: 34.5529x; 1.3323x over previous
import functools

import jax
import jax.numpy as jnp
from jax import lax
from jax.experimental import pallas as pl
from jax.experimental.pallas import tpu as pltpu
from jax.experimental.pallas import tpu_sc as plsc

N = 10000
E = 320000
C = 128

NC = 2
NS = 16
NW = NC * NS
EPW = E // NW
B = 64
NBF = EPW // B
TAIL = EPW - NBF * B
ZR = 1000
LG = C // 16

TC_BLK = 1000
TC_GRID = N // TC_BLK


def _proj_body(x_ref, w_ref, att2_ref, h_ref, asd_ref):
    h = jnp.dot(x_ref[...], w_ref[...], preferred_element_type=jnp.float32)
    h_ref[...] = h
    asd_ref[...] = jnp.dot(h, att2_ref[...], preferred_element_type=jnp.float32)


def _proj(x, w, att2):
    return pl.pallas_call(
        _proj_body,
        grid=(TC_GRID,),
        in_specs=[
            pl.BlockSpec((TC_BLK, C), lambda i: (i, 0)),
            pl.BlockSpec((C, C), lambda i: (0, 0)),
            pl.BlockSpec((C, 2), lambda i: (0, 0)),
        ],
        out_specs=[
            pl.BlockSpec((TC_BLK, C), lambda i: (i, 0)),
            pl.BlockSpec((TC_BLK, 2), lambda i: (i, 0)),
        ],
        out_shape=[
            jax.ShapeDtypeStruct((N, C), jnp.float32),
            jax.ShapeDtypeStruct((N, 2), jnp.float32),
        ],
    )(x, w, att2)


def _combine_temp(acc_ref, dent_ref, asd_ref, h_ref, bias_ref):
    a_s = asd_ref[:, 0:1]
    a_d = asd_ref[:, 1:2]
    es = a_s + a_d
    es = jnp.where(es >= 0, es, 0.2 * es)
    exs = jnp.exp(es)
    den = jnp.sum(dent_ref[...], axis=1, keepdims=True) + exs + 1e-16
    num = acc_ref[0] + acc_ref[1] + exs * h_ref[...]
    return num / den + bias_ref[...]


def _combine_mid_body(acc_ref, dent_ref, asd_ref, h_ref, bias_ref, w_ref,
                      att2_ref, h2_ref, asd2_ref):
    temp = _combine_temp(acc_ref, dent_ref, asd_ref, h_ref, bias_ref)
    y = jnp.where(temp >= 0, temp, 0.01 * temp) + temp
    h2 = jnp.dot(y, w_ref[...], preferred_element_type=jnp.float32)
    h2_ref[...] = h2
    asd2_ref[...] = jnp.dot(h2, att2_ref[...], preferred_element_type=jnp.float32)


def _combine_mid(acc, dent, asd, h, bias2, w, att2):
    return pl.pallas_call(
        _combine_mid_body,
        grid=(TC_GRID,),
        in_specs=[
            pl.BlockSpec((2, TC_BLK, C), lambda i: (0, i, 0)),
            pl.BlockSpec((TC_BLK, NW), lambda i: (i, 0)),
            pl.BlockSpec((TC_BLK, 2), lambda i: (i, 0)),
            pl.BlockSpec((TC_BLK, C), lambda i: (i, 0)),
            pl.BlockSpec((1, C), lambda i: (0, 0)),
            pl.BlockSpec((C, C), lambda i: (0, 0)),
            pl.BlockSpec((C, 2), lambda i: (0, 0)),
        ],
        out_specs=[
            pl.BlockSpec((TC_BLK, C), lambda i: (i, 0)),
            pl.BlockSpec((TC_BLK, 2), lambda i: (i, 0)),
        ],
        out_shape=[
            jax.ShapeDtypeStruct((N, C), jnp.float32),
            jax.ShapeDtypeStruct((N, 2), jnp.float32),
        ],
    )(acc, dent, asd, h, bias2, w, att2)


def _combine_final_body(acc_ref, dent_ref, asd_ref, h_ref, bias_ref, out_ref):
    temp = _combine_temp(acc_ref, dent_ref, asd_ref, h_ref, bias_ref)
    out_ref[...] = jnp.where(temp >= 0, temp, 0.01 * temp)


def _combine_final(acc, dent, asd, h, bias2):
    return pl.pallas_call(
        _combine_final_body,
        grid=(TC_GRID,),
        in_specs=[
            pl.BlockSpec((2, TC_BLK, C), lambda i: (0, i, 0)),
            pl.BlockSpec((TC_BLK, NW), lambda i: (i, 0)),
            pl.BlockSpec((TC_BLK, 2), lambda i: (i, 0)),
            pl.BlockSpec((TC_BLK, C), lambda i: (i, 0)),
            pl.BlockSpec((1, C), lambda i: (0, 0)),
        ],
        out_specs=pl.BlockSpec((TC_BLK, C), lambda i: (i, 0)),
        out_shape=jax.ShapeDtypeStruct((N, C), jnp.float32),
    )(acc, dent, asd, h, bias2)


_SC_MESH = plsc.VectorSubcoreMesh(core_axis_name="c", subcore_axis_name="s")


@functools.partial(
    pl.kernel,
    out_type=[
        jax.ShapeDtypeStruct((NC, N, C), jnp.float32),
        jax.ShapeDtypeStruct((NW * N,), jnp.float32),
    ],
    mesh=_SC_MESH,
    compiler_params=pltpu.CompilerParams(needs_layout_passes=False),
    scratch_types=[
        pltpu.VMEM((B,), jnp.int32),
        pltpu.VMEM((B,), jnp.int32),
        pltpu.VMEM((B,), jnp.int32),
        pltpu.VMEM((B,), jnp.int32),
        pltpu.VMEM((TAIL,), jnp.int32),
        pltpu.VMEM((TAIL,), jnp.int32),
        pltpu.VMEM((N,), jnp.float32),
        pltpu.VMEM((N,), jnp.float32),
        pltpu.VMEM((N,), jnp.float32),
        pltpu.VMEM((B, C), jnp.float32),
        pltpu.VMEM((B, C), jnp.float32),
        pltpu.VMEM_SHARED((N, C), jnp.float32),
        pltpu.SemaphoreType.DMA,
        pltpu.SemaphoreType.DMA,
        pltpu.SemaphoreType.DMA,
        pltpu.SemaphoreType.DMA,
    ],
)
def _sc_edges(h_hbm, a_s_hbm, a_d_hbm, src_hbm, dst_hbm,
              zacc_hbm, acc_out, den_out,
              src0, dst0, src1, dst1, srct, dstt, as_v, ad_v, den_v, rows0,
              rows1,
              acc_sh, gsem0, gsem1, ssem0, ssem1):
    c = lax.axis_index("c")
    s = lax.axis_index("s")
    wid = c * NS + s

    pltpu.sync_copy(a_s_hbm, as_v)
    pltpu.sync_copy(a_d_hbm, ad_v)

    @pl.when(s < N // ZR)
    def _():
        pltpu.sync_copy(zacc_hbm.at[pl.ds(s * ZR, ZR)],
                        acc_sh.at[pl.ds(s * ZR, ZR)])

    zero16 = jnp.zeros((16,), jnp.float32)

    @pl.loop(0, N // 16)
    def _(i):
        den_v[pl.ds(i * 16, 16)] = zero16

    plsc.subcore_barrier()

    def stage(b, src_b, dst_b, rows_v, gsem):
        base = wid * EPW + b * B
        pltpu.sync_copy(src_hbm.at[pl.ds(base, B)], src_b)
        pltpu.sync_copy(dst_hbm.at[pl.ds(base, B)], dst_b)
        pltpu.async_copy(h_hbm.at[src_b], rows_v, gsem)

    def compute(src_b, dst_b, rows_v, ngroups=B // 16):
        for g in range(ngroups):
            off = g * 16
            s16 = src_b[pl.ds(off, 16)]
            d16 = dst_b[pl.ds(off, 16)]
            e = plsc.load_gather(as_v, [s16]) + plsc.load_gather(ad_v, [d16])
            e = jnp.where(e >= 0, e, 0.2 * e)
            ex = jnp.exp(e)
            plsc.addupdate_scatter(den_v, [d16], ex)
            for j in range(16):
                r = g * 16 + j
                w = ex[j]
                for lg in range(LG):
                    rows_v[r, pl.ds(lg * 16, 16)] = (
                        rows_v[r, pl.ds(lg * 16, 16)] * w)

    def wait_gather(src_b, rows_v, gsem):
        pltpu.make_async_copy(h_hbm.at[src_b], rows_v, gsem).wait()

    def start_scatter(dst_b, rows_v, ssem):
        pltpu.async_copy(rows_v, acc_sh.at[dst_b], ssem, add=True)

    def wait_scatter(dst_b, rows_v, ssem):
        pltpu.make_async_copy(rows_v, acc_sh.at[dst_b], ssem).wait()

    stage(0, src0, dst0, rows0, gsem0)
    stage(1, src1, dst1, rows1, gsem1)

    @pl.loop(0, NBF // 2)
    def _(k):
        b0 = 2 * k
        wait_gather(src0, rows0, gsem0)
        compute(src0, dst0, rows0)
        start_scatter(dst0, rows0, ssem0)
        wait_gather(src1, rows1, gsem1)
        compute(src1, dst1, rows1)
        start_scatter(dst1, rows1, ssem1)

        @pl.when(b0 + 2 < NBF)
        def _():
            wait_scatter(dst0, rows0, ssem0)
            stage(b0 + 2, src0, dst0, rows0, gsem0)

        @pl.when(b0 + 3 < NBF)
        def _():
            wait_scatter(dst1, rows1, ssem1)
            stage(b0 + 3, src1, dst1, rows1, gsem1)

    wait_scatter(dst0, rows0, ssem0)
    wait_scatter(dst1, rows1, ssem1)

    tbase = wid * EPW + NBF * B
    pltpu.sync_copy(src_hbm.at[pl.ds(tbase, TAIL)], srct)
    pltpu.sync_copy(dst_hbm.at[pl.ds(tbase, TAIL)], dstt)
    rowst = rows0.at[pl.ds(0, TAIL)]
    pltpu.async_copy(h_hbm.at[srct], rowst, gsem0).wait()
    compute(srct, dstt, rows0, ngroups=TAIL // 16)
    pltpu.async_copy(rowst, acc_sh.at[dstt], ssem0, add=True).wait()

    pltpu.sync_copy(den_v, den_out.at[pl.ds(wid * N, N)])

    plsc.subcore_barrier()

    @pl.when(s < N // ZR)
    def _():
        pltpu.sync_copy(acc_sh.at[pl.ds(s * ZR, ZR)],
                        acc_out.at[c, pl.ds(s * ZR, ZR)])


def kernel(x, edges, W, att_src, att_dst, bias):
    att2 = jnp.stack([att_src, att_dst], axis=1)
    bias2 = bias.reshape(1, C)
    src_f = edges[0]
    dst_f = edges[1]
    zacc = jnp.zeros((N, C), jnp.float32)

    h1, asd1 = _proj(x, W, att2)
    acc1, den1 = _sc_edges(h1, asd1[:, 0], asd1[:, 1], src_f, dst_f, zacc)
    h2, asd2 = _combine_mid(acc1, den1.reshape(NW, N).T, asd1, h1, bias2, W,
                            att2)
    acc2, den2 = _sc_edges(h2, asd2[:, 0], asd2[:, 1], src_f, dst_f, zacc)
    return _combine_final(acc2, den2.reshape(NW, N).T, asd2, h2, bias2)


# speedup vs baseline: 39.6104x; 1.1464x over previous
import functools

import jax
import jax.numpy as jnp
from jax import lax
from jax.experimental import pallas as pl
from jax.experimental.pallas import tpu as pltpu
from jax.experimental.pallas import tpu_sc as plsc

N = 10000
E = 320000
C = 128

NC = 2
NS = 16
NW = NC * NS
EPW = E // NW
B = 64
NBF = EPW // B
TAIL = EPW - NBF * B
ZR = 1000
LG = C // 16

TC_BLK = 1000
TC_GRID = N // TC_BLK


def _proj_body(x_ref, w_ref, att2_ref, h_ref, asd_ref):
    h = jnp.dot(x_ref[...], w_ref[...], preferred_element_type=jnp.float32)
    h_ref[...] = h
    asd_ref[...] = jnp.dot(h, att2_ref[...], preferred_element_type=jnp.float32)


def _proj(x, w, att2):
    return pl.pallas_call(
        _proj_body,
        grid=(TC_GRID,),
        in_specs=[
            pl.BlockSpec((TC_BLK, C), lambda i: (i, 0)),
            pl.BlockSpec((C, C), lambda i: (0, 0)),
            pl.BlockSpec((C, 2), lambda i: (0, 0)),
        ],
        out_specs=[
            pl.BlockSpec((TC_BLK, C), lambda i: (i, 0)),
            pl.BlockSpec((TC_BLK, 2), lambda i: (i, 0)),
        ],
        out_shape=[
            jax.ShapeDtypeStruct((N, C), jnp.float32),
            jax.ShapeDtypeStruct((N, 2), jnp.float32),
        ],
    )(x, w, att2)


def _combine_temp(acc_ref, dent_ref, asd_ref, h_ref, bias_ref):
    a_s = asd_ref[:, 0:1]
    a_d = asd_ref[:, 1:2]
    es = a_s + a_d
    es = jnp.where(es >= 0, es, 0.2 * es)
    exs = jnp.exp(es)
    den = jnp.sum(dent_ref[...], axis=1, keepdims=True) + exs + 1e-16
    num = acc_ref[0] + acc_ref[1] + exs * h_ref[...]
    return num / den + bias_ref[...]


def _combine_mid_body(acc_ref, dent_ref, asd_ref, h_ref, bias_ref, w_ref,
                      att2_ref, h2_ref, asd2_ref):
    temp = _combine_temp(acc_ref, dent_ref, asd_ref, h_ref, bias_ref)
    y = jnp.where(temp >= 0, temp, 0.01 * temp) + temp
    h2 = jnp.dot(y, w_ref[...], preferred_element_type=jnp.float32)
    h2_ref[...] = h2
    asd2_ref[...] = jnp.dot(h2, att2_ref[...], preferred_element_type=jnp.float32)


def _combine_mid(acc, dent, asd, h, bias2, w, att2):
    return pl.pallas_call(
        _combine_mid_body,
        grid=(TC_GRID,),
        in_specs=[
            pl.BlockSpec((2, TC_BLK, C), lambda i: (0, i, 0)),
            pl.BlockSpec((TC_BLK, NW), lambda i: (i, 0)),
            pl.BlockSpec((TC_BLK, 2), lambda i: (i, 0)),
            pl.BlockSpec((TC_BLK, C), lambda i: (i, 0)),
            pl.BlockSpec((1, C), lambda i: (0, 0)),
            pl.BlockSpec((C, C), lambda i: (0, 0)),
            pl.BlockSpec((C, 2), lambda i: (0, 0)),
        ],
        out_specs=[
            pl.BlockSpec((TC_BLK, C), lambda i: (i, 0)),
            pl.BlockSpec((TC_BLK, 2), lambda i: (i, 0)),
        ],
        out_shape=[
            jax.ShapeDtypeStruct((N, C), jnp.float32),
            jax.ShapeDtypeStruct((N, 2), jnp.float32),
        ],
    )(acc, dent, asd, h, bias2, w, att2)


def _combine_final_body(acc_ref, dent_ref, asd_ref, h_ref, bias_ref, out_ref):
    temp = _combine_temp(acc_ref, dent_ref, asd_ref, h_ref, bias_ref)
    out_ref[...] = jnp.where(temp >= 0, temp, 0.01 * temp)


def _combine_final(acc, dent, asd, h, bias2):
    return pl.pallas_call(
        _combine_final_body,
        grid=(TC_GRID,),
        in_specs=[
            pl.BlockSpec((2, TC_BLK, C), lambda i: (0, i, 0)),
            pl.BlockSpec((TC_BLK, NW), lambda i: (i, 0)),
            pl.BlockSpec((TC_BLK, 2), lambda i: (i, 0)),
            pl.BlockSpec((TC_BLK, C), lambda i: (i, 0)),
            pl.BlockSpec((1, C), lambda i: (0, 0)),
        ],
        out_specs=pl.BlockSpec((TC_BLK, C), lambda i: (i, 0)),
        out_shape=jax.ShapeDtypeStruct((N, C), jnp.float32),
    )(acc, dent, asd, h, bias2)


_SC_MESH = plsc.VectorSubcoreMesh(core_axis_name="c", subcore_axis_name="s")


@functools.partial(
    pl.kernel,
    out_type=[
        jax.ShapeDtypeStruct((NC, N, C), jnp.float32),
        jax.ShapeDtypeStruct((NW * N,), jnp.float32),
    ],
    mesh=_SC_MESH,
    compiler_params=pltpu.CompilerParams(needs_layout_passes=False),
    scratch_types=[
        pltpu.VMEM((2, B), jnp.int32),
        pltpu.VMEM((2, B), jnp.int32),
        pltpu.VMEM((2, TAIL), jnp.int32),
        pltpu.VMEM((N,), jnp.float32),
        pltpu.VMEM((N,), jnp.float32),
        pltpu.VMEM((N,), jnp.float32),
        pltpu.VMEM((B, C), jnp.float32),
        pltpu.VMEM((B, C), jnp.float32),
        pltpu.VMEM_SHARED((N, C), jnp.float32),
        pltpu.SemaphoreType.DMA,
        pltpu.SemaphoreType.DMA,
        pltpu.SemaphoreType.DMA,
        pltpu.SemaphoreType.DMA,
    ],
)
def _sc_edges(h_hbm, a_s_hbm, a_d_hbm, er_hbm, ert_hbm,
              zacc_hbm, acc_out, den_out,
              idx0, idx1, idxt, as_v, ad_v, den_v, rows0, rows1,
              acc_sh, gsem0, gsem1, ssem0, ssem1):
    c = lax.axis_index("c")
    s = lax.axis_index("s")
    wid = c * NS + s

    pltpu.sync_copy(a_s_hbm, as_v)
    pltpu.sync_copy(a_d_hbm, ad_v)

    @pl.when(s < N // ZR)
    def _():
        pltpu.sync_copy(zacc_hbm.at[pl.ds(s * ZR, ZR)],
                        acc_sh.at[pl.ds(s * ZR, ZR)])

    zero16 = jnp.zeros((16,), jnp.float32)

    @pl.loop(0, N // 16)
    def _(i):
        den_v[pl.ds(i * 16, 16)] = zero16

    plsc.subcore_barrier()

    def stage(b, idx_b, rows_v, gsem):
        pltpu.sync_copy(er_hbm.at[wid, b], idx_b)
        pltpu.async_copy(h_hbm.at[idx_b.at[0]], rows_v, gsem)

    def compute(idx_b, rows_v, ngroups=B // 16):
        for g in range(ngroups):
            off = g * 16
            s16 = idx_b[0, pl.ds(off, 16)]
            d16 = idx_b[1, pl.ds(off, 16)]
            e = plsc.load_gather(as_v, [s16]) + plsc.load_gather(ad_v, [d16])
            e = jnp.where(e >= 0, e, 0.2 * e)
            ex = jnp.exp(e)
            plsc.addupdate_scatter(den_v, [d16], ex)
            for j in range(16):
                r = g * 16 + j
                w = ex[j]
                for lg in range(LG):
                    rows_v[r, pl.ds(lg * 16, 16)] = (
                        rows_v[r, pl.ds(lg * 16, 16)] * w)

    def wait_gather(idx_b, rows_v, gsem):
        pltpu.make_async_copy(h_hbm.at[idx_b.at[0]], rows_v, gsem).wait()

    def start_scatter(idx_b, rows_v, ssem):
        pltpu.async_copy(rows_v, acc_sh.at[idx_b.at[1]], ssem, add=True)

    def wait_scatter(idx_b, rows_v, ssem):
        pltpu.make_async_copy(rows_v, acc_sh.at[idx_b.at[1]], ssem).wait()

    stage(0, idx0, rows0, gsem0)
    stage(1, idx1, rows1, gsem1)

    @pl.loop(0, NBF // 2)
    def _(k):
        b0 = 2 * k
        wait_gather(idx0, rows0, gsem0)
        compute(idx0, rows0)
        start_scatter(idx0, rows0, ssem0)
        wait_gather(idx1, rows1, gsem1)
        compute(idx1, rows1)
        start_scatter(idx1, rows1, ssem1)

        @pl.when(b0 + 2 < NBF)
        def _():
            wait_scatter(idx0, rows0, ssem0)
            stage(b0 + 2, idx0, rows0, gsem0)

        @pl.when(b0 + 3 < NBF)
        def _():
            wait_scatter(idx1, rows1, ssem1)
            stage(b0 + 3, idx1, rows1, gsem1)

    wait_scatter(idx0, rows0, ssem0)
    wait_scatter(idx1, rows1, ssem1)

    pltpu.sync_copy(ert_hbm.at[wid], idxt)
    rowst = rows0.at[pl.ds(0, TAIL)]
    pltpu.async_copy(h_hbm.at[idxt.at[0]], rowst, gsem0).wait()
    compute(idxt, rows0, ngroups=TAIL // 16)
    pltpu.async_copy(rowst, acc_sh.at[idxt.at[1]], ssem0, add=True).wait()

    pltpu.sync_copy(den_v, den_out.at[pl.ds(wid * N, N)])

    plsc.subcore_barrier()

    @pl.when(s < N // ZR)
    def _():
        pltpu.sync_copy(acc_sh.at[pl.ds(s * ZR, ZR)],
                        acc_out.at[c, pl.ds(s * ZR, ZR)])


def kernel(x, edges, W, att_src, att_dst, bias):
    att2 = jnp.stack([att_src, att_dst], axis=1)
    bias2 = bias.reshape(1, C)
    src_c = edges[0].reshape(NW, EPW)
    dst_c = edges[1].reshape(NW, EPW)
    er = jnp.stack([src_c[:, :NBF * B].reshape(NW, NBF, B),
                    dst_c[:, :NBF * B].reshape(NW, NBF, B)], axis=2)
    ert = jnp.stack([src_c[:, NBF * B:], dst_c[:, NBF * B:]], axis=1)
    zacc = jnp.zeros((N, C), jnp.float32)

    h1, asd1 = _proj(x, W, att2)
    acc1, den1 = _sc_edges(h1, asd1[:, 0], asd1[:, 1], er, ert, zacc)
    h2, asd2 = _combine_mid(acc1, den1.reshape(NW, N).T, asd1, h1, bias2, W,
                            att2)
    acc2, den2 = _sc_edges(h2, asd2[:, 0], asd2[:, 1], er, ert, zacc)
    return _combine_final(acc2, den2.reshape(NW, N).T, asd2, h2, bias2)
